# baseline (speedup 1.0000x reference)
import jax
import jax.numpy as jnp
from jax.experimental import pallas as pl
from jax.experimental.pallas import tpu as pltpu

_NCHUNK = 2


def _agg_kernel(x_hbm, len_ref, spans_ref, W_ref, b_ref, out_hbm,
                xbuf, obuf, in_sems, out_sems):
    B, T, D = x_hbm.shape
    L = spans_ref.shape[1]
    Tc = T // _NCHUNK

    for ci in range(B * _NCHUNK):
        bi, c = divmod(ci, _NCHUNK)
        pltpu.make_async_copy(
            x_hbm.at[bi, pl.ds(c * Tc, Tc)],
            xbuf.at[bi, pl.ds(c * Tc, Tc)],
            in_sems.at[ci],
        ).start()

    Wt = W_ref[...].T.astype(jnp.bfloat16)
    bias = b_ref[...]

    for bi in range(B):
        ii = spans_ref[bi, :, 0:1]
        jj = spans_ref[bi, :, 1:2]
        width = (jj - ii).astype(jnp.float32)
        j_iota = jax.lax.broadcasted_iota(jnp.int32, (L, 1), 0)
        valid = (j_iota < len_ref[bi]).astype(jnp.float32)
        scale = valid / width
        agg = None
        for c in range(_NCHUNK):
            pltpu.make_async_copy(
                x_hbm.at[bi, pl.ds(c * Tc, Tc)],
                xbuf.at[bi, pl.ds(c * Tc, Tc)],
                in_sems.at[bi * _NCHUNK + c],
            ).wait()
            t = c * Tc + jax.lax.broadcasted_iota(jnp.int32, (L, Tc), 1)
            mask = (t >= ii) & (t < jj)
            M = jnp.where(mask, scale, 0.0)
            part = jnp.dot(
                M.astype(jnp.bfloat16),
                xbuf[bi, c * Tc:(c + 1) * Tc].astype(jnp.bfloat16),
                preferred_element_type=jnp.float32,
            )
            agg = part if agg is None else agg + part
        obuf[bi] = (
            jnp.dot(agg.astype(jnp.bfloat16), Wt,
                    preferred_element_type=jnp.float32)
            + bias
        )
        pltpu.make_async_copy(obuf.at[bi], out_hbm.at[bi],
                              out_sems.at[bi]).start()

    for bi in range(B):
        pltpu.make_async_copy(obuf.at[bi], out_hbm.at[bi],
                              out_sems.at[bi]).wait()


def kernel(input, lengths, span_indexes, W, b):
    B, T, D = input.shape
    L = span_indexes.shape[1]

    out = pl.pallas_call(
        _agg_kernel,
        in_specs=[
            pl.BlockSpec(memory_space=pltpu.MemorySpace.HBM),
            pl.BlockSpec(memory_space=pltpu.SMEM),
            pl.BlockSpec((B, L, 2), lambda: (0, 0, 0)),
            pl.BlockSpec((D, D), lambda: (0, 0)),
            pl.BlockSpec((1, D), lambda: (0, 0)),
        ],
        out_specs=pl.BlockSpec(memory_space=pltpu.MemorySpace.HBM),
        out_shape=jax.ShapeDtypeStruct((B, L, D), jnp.float32),
        scratch_shapes=[
            pltpu.VMEM((B, T, D), jnp.float32),
            pltpu.VMEM((B, L, D), jnp.float32),
            pltpu.SemaphoreType.DMA((B * _NCHUNK,)),
            pltpu.SemaphoreType.DMA((B,)),
        ],
    )(input, lengths, span_indexes, W, b.reshape(1, D))
    return out
